# baseline (speedup 1.0000x reference)
import functools

import jax
import jax.numpy as jnp
import numpy as np
from jax import lax
from jax.experimental import pallas as pl
from jax.experimental.pallas import tpu as pltpu
from jax.experimental.pallas import tpu_sc as plsc

N = 10000
E = 320000
ET = E + N
NHEAD = 8
NHID = 16
HID = NHEAD * NHID
NLABEL = 4
FDIM = HID + NLABEL

NP = 10240
BLK = 1024
NW = 32
C = 48
G = 12
NGRP = 18
NCHUNK = G * NGRP
SUP = NCHUNK // 4
TPW = NCHUNK * C
EP = NW * TPW
PACK = HID + 16
ROWS_PER_TILE = NP // 16


def _dot(a, b):
    return jax.lax.dot(a, b, precision=jax.lax.Precision.HIGHEST,
                       preferred_element_type=jnp.float32)


def _elu(z):
    return jnp.where(z > 0, z, jnp.exp(jnp.minimum(z, 0.0)) - 1.0)


def _tc_temb_body(tsc_ref, freq_ref, tw1a_ref, tw1b_ref, tb1_ref,
                  tw2_ref, tb2_ref, temb_ref):
    emb = tsc_ref[...] * freq_ref[...]
    z = _dot(jnp.sin(emb), tw1a_ref[...]) + _dot(jnp.cos(emb), tw1b_ref[...]) + tb1_ref[...]
    temb_ref[...] = _dot(_elu(z), tw2_ref[...]) + tb2_ref[...]


def _tc_temb(tsc, freq, tw1, tb1, tw2, tb2):
    return pl.pallas_call(
        _tc_temb_body,
        grid=(NP // BLK,),
        in_specs=[
            pl.BlockSpec((BLK, 1), lambda i: (i, 0)),
            pl.BlockSpec((1, 64), lambda i: (0, 0)),
            pl.BlockSpec((64, 128), lambda i: (0, 0)),
            pl.BlockSpec((64, 128), lambda i: (0, 0)),
            pl.BlockSpec((128,), lambda i: (0,)),
            pl.BlockSpec((128, HID), lambda i: (0, 0)),
            pl.BlockSpec((HID,), lambda i: (0,)),
        ],
        out_specs=pl.BlockSpec((BLK, HID), lambda i: (i, 0)),
        out_shape=jax.ShapeDtypeStruct((NP, HID), jnp.float32),
    )(tsc, freq, tw1[:64], tw1[64:], tb1, tw2, tb2)


def _tc_pre_body(x_ref, q_ref, w0x_ref, w0q_ref, af_ref, df_ref, s_ref,
                 hal_ref, ad_ref):
    h = _dot(x_ref[...], w0x_ref[...]) + _dot(q_ref[...], w0q_ref[...])
    hal_ref[:, :HID] = h
    hal_ref[:, HID:] = _dot(h * af_ref[...], s_ref[...])
    ad_ref[...] = _dot(h * df_ref[...], s_ref[...])


def _tc_pre(xp, qp, W0, asrc, adst, S):
    return pl.pallas_call(
        _tc_pre_body,
        grid=(NP // BLK,),
        in_specs=[
            pl.BlockSpec((BLK, 128), lambda i: (i, 0)),
            pl.BlockSpec((BLK, NLABEL), lambda i: (i, 0)),
            pl.BlockSpec((128, HID), lambda i: (0, 0)),
            pl.BlockSpec((NLABEL, HID), lambda i: (0, 0)),
            pl.BlockSpec((1, HID), lambda i: (0, 0)),
            pl.BlockSpec((1, HID), lambda i: (0, 0)),
            pl.BlockSpec((HID, 16), lambda i: (0, 0)),
        ],
        out_specs=[
            pl.BlockSpec((BLK, PACK), lambda i: (i, 0)),
            pl.BlockSpec((BLK, 16), lambda i: (i, 0)),
        ],
        out_shape=[
            jax.ShapeDtypeStruct((NP, PACK), jnp.float32),
            jax.ShapeDtypeStruct((NP, 16), jnp.float32),
        ],
    )(xp, qp, W0[:128], W0[128:], asrc.reshape(1, HID), adst.reshape(1, HID), S)


_SC_MESH = plsc.VectorSubcoreMesh(core_axis_name="c", subcore_axis_name="s")


@functools.partial(
    pl.kernel,
    out_type=[jax.ShapeDtypeStruct((2, NP, HID), jnp.float32),
              jax.ShapeDtypeStruct((2, NP, 16), jnp.float32)],
    mesh=_SC_MESH,
    scratch_types=[
        pltpu.VMEM((2, G, C), jnp.int32),
        pltpu.VMEM((2, G, C), jnp.int32),
        pltpu.VMEM((4, C, PACK), jnp.float32),
        pltpu.VMEM((4, C, 16), jnp.float32),
        pltpu.VMEM_SHARED((NP, PACK), jnp.float32),
        [pltpu.SemaphoreType.DMA] * 4,
        [pltpu.SemaphoreType.DMA] * 4,
        [pltpu.SemaphoreType.DMA] * 4,
        [pltpu.SemaphoreType.DMA] * 2,
    ],
    compiler_params=pltpu.CompilerParams(use_tc_tiling_on_sc=False),
)
def _sc_gat(hal_hbm, ad_hbm, srcp_hbm, dstp_hbm, zero_hbm, out_hbm, den_hbm,
            sidx, didx, hal, ad, acc_sh, hsem, asem, ssem, isem):
    c = lax.axis_index("c")
    s = lax.axis_index("s")
    wid = c * 16 + s
    pltpu.sync_copy(zero_hbm.at[pl.ds(s * ROWS_PER_TILE, ROWS_PER_TILE)],
                    acc_sh.at[pl.ds(s * ROWS_PER_TILE, ROWS_PER_TILE)])
    plsc.subcore_barrier()

    lane = lax.iota(jnp.int32, 16)

    def wait_gathers(b):
        pltpu.make_async_copy(hal_hbm.at[pl.ds(0, C)], hal.at[b], hsem[b]).wait()
        pltpu.make_async_copy(ad_hbm.at[pl.ds(0, C)], ad.at[b], asem[b]).wait()

    def wait_scatter(b):
        pltpu.make_async_copy(hal.at[b], acc_sh.at[pl.ds(0, C)], ssem[b]).wait()

    def issue_gather(kk, b):
        gi = (kk // G) % 2
        r = kk % G
        pltpu.async_copy(hal_hbm.at[sidx.at[gi].at[r]], hal.at[b], hsem[b])
        pltpu.async_copy(ad_hbm.at[didx.at[gi].at[r]], ad.at[b], asem[b])

    def stage_group(g, gi):
        pltpu.async_copy(srcp_hbm.at[wid].at[g], sidx.at[gi], isem[gi])
        pltpu.async_copy(dstp_hbm.at[wid].at[g], didx.at[gi], isem[gi])

    def wait_stage(gi):
        pltpu.make_async_copy(srcp_hbm.at[wid].at[0], sidx.at[gi], isem[gi]).wait()
        pltpu.make_async_copy(dstp_hbm.at[wid].at[0], didx.at[gi], isem[gi]).wait()

    def compute(kk, b):
        def edge(e, carry2):
            z = hal[b, e, pl.ds(HID, 16)] + ad[b, e]
            z = jnp.maximum(z, 0.2 * z)
            ee = jnp.exp(z)
            ee = jnp.where(lane < NHEAD, ee, 0.0)
            hal[b, e, pl.ds(HID, 16)] = ee
            for hd in range(NHEAD):
                hal[b, e, pl.ds(hd * 16, 16)] = ee[hd] * hal[b, e, pl.ds(hd * 16, 16)]
            return carry2

        lax.fori_loop(0, C, edge, 0, unroll=4)
        gi = (kk // G) % 2
        r = kk % G
        pltpu.async_copy(hal.at[b], acc_sh.at[didx.at[gi].at[r]], ssem[b], add=True)

    pltpu.sync_copy(srcp_hbm.at[wid].at[0], sidx.at[0])
    pltpu.sync_copy(dstp_hbm.at[wid].at[0], didx.at[0])
    stage_group(1, 1)
    issue_gather(0, 0)
    issue_gather(1, 1)

    def super_chunk(sup, carry):
        kk0 = sup * 4
        grp = sup // 3
        for i in range(4):
            kk = kk0 + i
            b = i
            if i == 0:
                @pl.when((sup % 3 == 0) & (grp + 1 < NGRP))
                def _():
                    @pl.when(grp % 2 == 0)
                    def _():
                        stage_group(grp + 1, 1)
                    @pl.when(grp % 2 == 1)
                    def _():
                        stage_group(grp + 1, 0)
            if i == 2:
                @pl.when((sup % 3 == 2) & (sup < SUP - 1))
                def _():
                    @pl.when((grp + 1) % 2 == 0)
                    def _():
                        wait_stage(0)
                    @pl.when((grp + 1) % 2 == 1)
                    def _():
                        wait_stage(1)

            @pl.when(kk >= 2)
            def _():
                wait_scatter((b + 2) % 4)

            @pl.when(kk + 2 < NCHUNK)
            def _():
                issue_gather(kk + 2, (b + 2) % 4)

            wait_gathers(b)
            compute(kk, b)
        return carry

    lax.fori_loop(0, SUP, super_chunk, 0)
    wait_scatter(2)
    wait_scatter(3)
    plsc.subcore_barrier()
    rs = pl.ds(s * ROWS_PER_TILE, ROWS_PER_TILE)
    pltpu.sync_copy(acc_sh.at[rs, pl.ds(0, HID)], out_hbm.at[c].at[rs])
    pltpu.sync_copy(acc_sh.at[rs, pl.ds(HID, 16)], den_hbm.at[c].at[rs])


def _tc_mid_body(o0_ref, o1_ref, d0_ref, d1_ref, temb_ref, q_ref, b_ref,
                 w1h_ref, w1q_ref, af_ref, df_ref, s_ref, b16_ref,
                 hal_ref, ad_ref):
    denb = _dot(d0_ref[...] + d1_ref[...], b16_ref[...]) + 1e-16
    g = (o0_ref[...] + o1_ref[...]) / denb + b_ref[...]
    hpre = _elu(g + temb_ref[...])
    h = _dot(hpre, w1h_ref[...]) + _dot(q_ref[...], w1q_ref[...])
    hal_ref[:, :HID] = h
    hal_ref[:, HID:] = _dot(h * af_ref[...], s_ref[...])
    ad_ref[...] = _dot(h * df_ref[...], s_ref[...])


def _tc_mid(o0, o1, d0, d1, temb, qp, b, W1, asrc, adst, S, B16):
    return pl.pallas_call(
        _tc_mid_body,
        grid=(NP // BLK,),
        in_specs=[
            pl.BlockSpec((BLK, HID), lambda i: (i, 0)),
            pl.BlockSpec((BLK, HID), lambda i: (i, 0)),
            pl.BlockSpec((BLK, 16), lambda i: (i, 0)),
            pl.BlockSpec((BLK, 16), lambda i: (i, 0)),
            pl.BlockSpec((BLK, HID), lambda i: (i, 0)),
            pl.BlockSpec((BLK, NLABEL), lambda i: (i, 0)),
            pl.BlockSpec((1, HID), lambda i: (0, 0)),
            pl.BlockSpec((HID, HID), lambda i: (0, 0)),
            pl.BlockSpec((NLABEL, HID), lambda i: (0, 0)),
            pl.BlockSpec((1, HID), lambda i: (0, 0)),
            pl.BlockSpec((1, HID), lambda i: (0, 0)),
            pl.BlockSpec((HID, 16), lambda i: (0, 0)),
            pl.BlockSpec((16, HID), lambda i: (0, 0)),
        ],
        out_specs=[
            pl.BlockSpec((BLK, PACK), lambda i: (i, 0)),
            pl.BlockSpec((BLK, 16), lambda i: (i, 0)),
        ],
        out_shape=[
            jax.ShapeDtypeStruct((NP, PACK), jnp.float32),
            jax.ShapeDtypeStruct((NP, 16), jnp.float32),
        ],
    )(o0, o1, d0, d1, temb, qp, b.reshape(1, HID), W1[:HID], W1[HID:],
      asrc.reshape(1, HID), adst.reshape(1, HID), S, B16)


def _tc_final_body(o0_ref, o1_ref, d0_ref, d1_ref, temb_ref, q_ref, b_ref,
                   fw1h_ref, fw1q_ref, fb1_ref, fw2_ref, fb2_ref, b16_ref,
                   out_ref):
    denb = _dot(d0_ref[...] + d1_ref[...], b16_ref[...]) + 1e-16
    g = (o0_ref[...] + o1_ref[...]) / denb + b_ref[...]
    hpre = _elu(g + temb_ref[...])
    z = _elu(_dot(hpre, fw1h_ref[...]) + _dot(q_ref[...], fw1q_ref[...]) + fb1_ref[...])
    out_ref[...] = _dot(z, fw2_ref[...]) + fb2_ref[...]


def _tc_final(o0, o1, d0, d1, temb, qp, b, fw1, fb1, fw2, fb2, B16):
    return pl.pallas_call(
        _tc_final_body,
        grid=(NP // BLK,),
        in_specs=[
            pl.BlockSpec((BLK, HID), lambda i: (i, 0)),
            pl.BlockSpec((BLK, HID), lambda i: (i, 0)),
            pl.BlockSpec((BLK, 16), lambda i: (i, 0)),
            pl.BlockSpec((BLK, 16), lambda i: (i, 0)),
            pl.BlockSpec((BLK, HID), lambda i: (i, 0)),
            pl.BlockSpec((BLK, NLABEL), lambda i: (i, 0)),
            pl.BlockSpec((1, HID), lambda i: (0, 0)),
            pl.BlockSpec((HID, 2 * FDIM), lambda i: (0, 0)),
            pl.BlockSpec((NLABEL, 2 * FDIM), lambda i: (0, 0)),
            pl.BlockSpec((2 * FDIM,), lambda i: (0,)),
            pl.BlockSpec((2 * FDIM, NLABEL), lambda i: (0, 0)),
            pl.BlockSpec((NLABEL,), lambda i: (0,)),
            pl.BlockSpec((16, HID), lambda i: (0, 0)),
        ],
        out_specs=pl.BlockSpec((BLK, NLABEL), lambda i: (i, 0)),
        out_shape=jax.ShapeDtypeStruct((NP, NLABEL), jnp.float32),
    )(o0, o1, d0, d1, temb, qp, b.reshape(1, HID), fw1[:HID], fw1[HID:],
      fb1, fw2, fb2, B16)


def kernel(x, q_Y_sample, adj, t, num_steps, W0, asrc0, adst0, b0,
           W1, asrc1, adst1, b1, tw1, tb1, tw2, tb2, fw1, fb1, fw2, fb2):
    f32 = jnp.float32
    xp = jnp.zeros((NP, 128), f32).at[:N].set(x)
    qp = jnp.zeros((NP, NLABEL), f32).at[:N].set(q_Y_sample)
    tsc = jnp.zeros((NP, 1), f32).at[:N, 0].set(t / num_steps * num_steps * 4.0)
    freq = jnp.exp(np.arange(64, dtype=np.float32) * (-(np.log(10000.0) / 63))
                   ).reshape(1, 64).astype(f32)
    hd_of = np.arange(HID) // NHID
    S = np.zeros((HID, 16), np.float32)
    S[np.arange(HID), hd_of] = 1.0
    B16 = jnp.asarray(S.T)
    S = jnp.asarray(S)
    loop = jnp.arange(N, dtype=adj.dtype)
    srcp = jnp.full((EP,), N, jnp.int32).at[:E].set(adj[0]).at[E:ET].set(loop)
    dstp = jnp.full((EP,), N, jnp.int32).at[:E].set(adj[1]).at[E:ET].set(loop)
    srcp = srcp.reshape(NW, NGRP, G, C)
    dstp = dstp.reshape(NW, NGRP, G, C)
    zero = jnp.zeros((NP, PACK), f32)

    temb = _tc_temb(tsc, freq, tw1, tb1, tw2, tb2)
    hal0, ad0 = _tc_pre(xp, qp, W0, asrc0, adst0, S)
    out0, den0 = _sc_gat(hal0, ad0, srcp, dstp, zero)
    hal1, ad1 = _tc_mid(out0[0], out0[1], den0[0], den0[1], temb, qp, b0,
                        W1, asrc1, adst1, S, B16)
    out1, den1 = _sc_gat(hal1, ad1, srcp, dstp, zero)
    out = _tc_final(out1[0], out1[1], den1[0], den1[1], temb, qp, b1,
                    fw1, fb1, fw2, fb2, B16)
    return out[:N]


# speedup vs baseline: 1.0011x; 1.0011x over previous
import functools

import jax
import jax.numpy as jnp
import numpy as np
from jax import lax
from jax.experimental import pallas as pl
from jax.experimental.pallas import tpu as pltpu
from jax.experimental.pallas import tpu_sc as plsc

N = 10000
E = 320000
ET = E + N
NHEAD = 8
NHID = 16
HID = NHEAD * NHID
NLABEL = 4
FDIM = HID + NLABEL

NP = 10240
BLK = 1024
NW = 32
C = 48
G = 12
NGRP = 18
NCHUNK = G * NGRP
SUP = NCHUNK // 4
TPW = NCHUNK * C
EP = NW * TPW
PACK = HID + 16
ROWS_PER_TILE = NP // 16


def _dot(a, b):
    return jax.lax.dot(a, b, precision=jax.lax.Precision.HIGHEST,
                       preferred_element_type=jnp.float32)


def _elu(z):
    return jnp.where(z > 0, z, jnp.exp(jnp.minimum(z, 0.0)) - 1.0)


def _tc_temb_body(tsc_ref, freq_ref, tw1a_ref, tw1b_ref, tb1_ref,
                  tw2_ref, tb2_ref, temb_ref):
    emb = tsc_ref[...] * freq_ref[...]
    z = _dot(jnp.sin(emb), tw1a_ref[...]) + _dot(jnp.cos(emb), tw1b_ref[...]) + tb1_ref[...]
    temb_ref[...] = _dot(_elu(z), tw2_ref[...]) + tb2_ref[...]


def _tc_temb(tsc, freq, tw1, tb1, tw2, tb2):
    return pl.pallas_call(
        _tc_temb_body,
        grid=(NP // BLK,),
        in_specs=[
            pl.BlockSpec((BLK, 1), lambda i: (i, 0)),
            pl.BlockSpec((1, 64), lambda i: (0, 0)),
            pl.BlockSpec((64, 128), lambda i: (0, 0)),
            pl.BlockSpec((64, 128), lambda i: (0, 0)),
            pl.BlockSpec((128,), lambda i: (0,)),
            pl.BlockSpec((128, HID), lambda i: (0, 0)),
            pl.BlockSpec((HID,), lambda i: (0,)),
        ],
        out_specs=pl.BlockSpec((BLK, HID), lambda i: (i, 0)),
        out_shape=jax.ShapeDtypeStruct((NP, HID), jnp.float32),
    )(tsc, freq, tw1[:64], tw1[64:], tb1, tw2, tb2)


def _tc_pre_body(x_ref, q_ref, w0x_ref, w0q_ref, af_ref, df_ref, s_ref,
                 hal_ref, ad_ref):
    h = _dot(x_ref[...], w0x_ref[...]) + _dot(q_ref[...], w0q_ref[...])
    hal_ref[:, :HID] = h
    hal_ref[:, HID:] = _dot(h * af_ref[...], s_ref[...])
    ad_ref[...] = _dot(h * df_ref[...], s_ref[...])


def _tc_pre(xp, qp, W0, asrc, adst, S):
    return pl.pallas_call(
        _tc_pre_body,
        grid=(NP // BLK,),
        in_specs=[
            pl.BlockSpec((BLK, 128), lambda i: (i, 0)),
            pl.BlockSpec((BLK, NLABEL), lambda i: (i, 0)),
            pl.BlockSpec((128, HID), lambda i: (0, 0)),
            pl.BlockSpec((NLABEL, HID), lambda i: (0, 0)),
            pl.BlockSpec((1, HID), lambda i: (0, 0)),
            pl.BlockSpec((1, HID), lambda i: (0, 0)),
            pl.BlockSpec((HID, 16), lambda i: (0, 0)),
        ],
        out_specs=[
            pl.BlockSpec((BLK, PACK), lambda i: (i, 0)),
            pl.BlockSpec((BLK, 16), lambda i: (i, 0)),
        ],
        out_shape=[
            jax.ShapeDtypeStruct((NP, PACK), jnp.float32),
            jax.ShapeDtypeStruct((NP, 16), jnp.float32),
        ],
    )(xp, qp, W0[:128], W0[128:], asrc.reshape(1, HID), adst.reshape(1, HID), S)


@functools.cache
def _make_sc_gat():
  mesh = plsc.VectorSubcoreMesh(core_axis_name="c", subcore_axis_name="s")

  @functools.partial(
      pl.kernel,
      out_type=[jax.ShapeDtypeStruct((2, NP, HID), jnp.float32),
                jax.ShapeDtypeStruct((2, NP, 16), jnp.float32)],
      mesh=mesh,
      scratch_types=[
          pltpu.VMEM((2, G, C), jnp.int32),
          pltpu.VMEM((2, G, C), jnp.int32),
          pltpu.VMEM((4, C, PACK), jnp.float32),
          pltpu.VMEM((4, C, 16), jnp.float32),
          pltpu.VMEM_SHARED((NP, PACK), jnp.float32),
          [pltpu.SemaphoreType.DMA] * 4,
          [pltpu.SemaphoreType.DMA] * 4,
          [pltpu.SemaphoreType.DMA] * 4,
          [pltpu.SemaphoreType.DMA] * 2,
      ],
      compiler_params=pltpu.CompilerParams(use_tc_tiling_on_sc=False),
  )
  def _sc_gat(hal_hbm, ad_hbm, srcp_hbm, dstp_hbm, zero_hbm, out_hbm, den_hbm,
              sidx, didx, hal, ad, acc_sh, hsem, asem, ssem, isem):
      c = lax.axis_index("c")
      s = lax.axis_index("s")
      wid = c * 16 + s
      pltpu.sync_copy(zero_hbm.at[pl.ds(s * ROWS_PER_TILE, ROWS_PER_TILE)],
                      acc_sh.at[pl.ds(s * ROWS_PER_TILE, ROWS_PER_TILE)])
      plsc.subcore_barrier()

      lane = lax.iota(jnp.int32, 16)

      def wait_gathers(b):
          pltpu.make_async_copy(hal_hbm.at[pl.ds(0, C)], hal.at[b], hsem[b]).wait()
          pltpu.make_async_copy(ad_hbm.at[pl.ds(0, C)], ad.at[b], asem[b]).wait()

      def wait_scatter(b):
          pltpu.make_async_copy(hal.at[b], acc_sh.at[pl.ds(0, C)], ssem[b]).wait()

      def issue_gather(kk, b):
          gi = (kk // G) % 2
          r = kk % G
          pltpu.async_copy(hal_hbm.at[sidx.at[gi].at[r]], hal.at[b], hsem[b])
          pltpu.async_copy(ad_hbm.at[didx.at[gi].at[r]], ad.at[b], asem[b])

      def stage_group(g, gi):
          pltpu.async_copy(srcp_hbm.at[wid].at[g], sidx.at[gi], isem[gi])
          pltpu.async_copy(dstp_hbm.at[wid].at[g], didx.at[gi], isem[gi])

      def wait_stage(gi):
          pltpu.make_async_copy(srcp_hbm.at[wid].at[0], sidx.at[gi], isem[gi]).wait()
          pltpu.make_async_copy(dstp_hbm.at[wid].at[0], didx.at[gi], isem[gi]).wait()

      def compute(kk, b):
          def edge(e, carry2):
              z = hal[b, e, pl.ds(HID, 16)] + ad[b, e]
              z = jnp.maximum(z, 0.2 * z)
              ee = jnp.exp(z)
              ee = jnp.where(lane < NHEAD, ee, 0.0)
              hal[b, e, pl.ds(HID, 16)] = ee
              for hd in range(NHEAD):
                  hal[b, e, pl.ds(hd * 16, 16)] = ee[hd] * hal[b, e, pl.ds(hd * 16, 16)]
              return carry2

          lax.fori_loop(0, C, edge, 0, unroll=4)
          gi = (kk // G) % 2
          r = kk % G
          pltpu.async_copy(hal.at[b], acc_sh.at[didx.at[gi].at[r]], ssem[b], add=True)

      pltpu.sync_copy(srcp_hbm.at[wid].at[0], sidx.at[0])
      pltpu.sync_copy(dstp_hbm.at[wid].at[0], didx.at[0])
      stage_group(1, 1)
      issue_gather(0, 0)
      issue_gather(1, 1)

      def super_chunk(sup, carry):
          kk0 = sup * 4
          grp = sup // 3
          for i in range(4):
              kk = kk0 + i
              b = i
              if i == 0:
                  @pl.when((sup % 3 == 0) & (grp + 1 < NGRP))
                  def _():
                      @pl.when(grp % 2 == 0)
                      def _():
                          stage_group(grp + 1, 1)
                      @pl.when(grp % 2 == 1)
                      def _():
                          stage_group(grp + 1, 0)
              if i == 2:
                  @pl.when((sup % 3 == 2) & (sup < SUP - 1))
                  def _():
                      @pl.when((grp + 1) % 2 == 0)
                      def _():
                          wait_stage(0)
                      @pl.when((grp + 1) % 2 == 1)
                      def _():
                          wait_stage(1)

              @pl.when(kk >= 2)
              def _():
                  wait_scatter((b + 2) % 4)

              @pl.when(kk + 2 < NCHUNK)
              def _():
                  issue_gather(kk + 2, (b + 2) % 4)

              wait_gathers(b)
              compute(kk, b)
          return carry

      lax.fori_loop(0, SUP, super_chunk, 0)
      wait_scatter(2)
      wait_scatter(3)
      plsc.subcore_barrier()
      rs = pl.ds(s * ROWS_PER_TILE, ROWS_PER_TILE)
      pltpu.sync_copy(acc_sh.at[rs, pl.ds(0, HID)], out_hbm.at[c].at[rs])
      pltpu.sync_copy(acc_sh.at[rs, pl.ds(HID, 16)], den_hbm.at[c].at[rs])

  return _sc_gat


def _tc_mid_body(o0_ref, o1_ref, d0_ref, d1_ref, temb_ref, q_ref, b_ref,
                 w1h_ref, w1q_ref, af_ref, df_ref, s_ref, b16_ref,
                 hal_ref, ad_ref):
    denb = _dot(d0_ref[...] + d1_ref[...], b16_ref[...]) + 1e-16
    g = (o0_ref[...] + o1_ref[...]) / denb + b_ref[...]
    hpre = _elu(g + temb_ref[...])
    h = _dot(hpre, w1h_ref[...]) + _dot(q_ref[...], w1q_ref[...])
    hal_ref[:, :HID] = h
    hal_ref[:, HID:] = _dot(h * af_ref[...], s_ref[...])
    ad_ref[...] = _dot(h * df_ref[...], s_ref[...])


def _tc_mid(o0, o1, d0, d1, temb, qp, b, W1, asrc, adst, S, B16):
    return pl.pallas_call(
        _tc_mid_body,
        grid=(NP // BLK,),
        in_specs=[
            pl.BlockSpec((BLK, HID), lambda i: (i, 0)),
            pl.BlockSpec((BLK, HID), lambda i: (i, 0)),
            pl.BlockSpec((BLK, 16), lambda i: (i, 0)),
            pl.BlockSpec((BLK, 16), lambda i: (i, 0)),
            pl.BlockSpec((BLK, HID), lambda i: (i, 0)),
            pl.BlockSpec((BLK, NLABEL), lambda i: (i, 0)),
            pl.BlockSpec((1, HID), lambda i: (0, 0)),
            pl.BlockSpec((HID, HID), lambda i: (0, 0)),
            pl.BlockSpec((NLABEL, HID), lambda i: (0, 0)),
            pl.BlockSpec((1, HID), lambda i: (0, 0)),
            pl.BlockSpec((1, HID), lambda i: (0, 0)),
            pl.BlockSpec((HID, 16), lambda i: (0, 0)),
            pl.BlockSpec((16, HID), lambda i: (0, 0)),
        ],
        out_specs=[
            pl.BlockSpec((BLK, PACK), lambda i: (i, 0)),
            pl.BlockSpec((BLK, 16), lambda i: (i, 0)),
        ],
        out_shape=[
            jax.ShapeDtypeStruct((NP, PACK), jnp.float32),
            jax.ShapeDtypeStruct((NP, 16), jnp.float32),
        ],
    )(o0, o1, d0, d1, temb, qp, b.reshape(1, HID), W1[:HID], W1[HID:],
      asrc.reshape(1, HID), adst.reshape(1, HID), S, B16)


def _tc_final_body(o0_ref, o1_ref, d0_ref, d1_ref, temb_ref, q_ref, b_ref,
                   fw1h_ref, fw1q_ref, fb1_ref, fw2_ref, fb2_ref, b16_ref,
                   out_ref):
    denb = _dot(d0_ref[...] + d1_ref[...], b16_ref[...]) + 1e-16
    g = (o0_ref[...] + o1_ref[...]) / denb + b_ref[...]
    hpre = _elu(g + temb_ref[...])
    z = _elu(_dot(hpre, fw1h_ref[...]) + _dot(q_ref[...], fw1q_ref[...]) + fb1_ref[...])
    out_ref[...] = _dot(z, fw2_ref[...]) + fb2_ref[...]


def _tc_final(o0, o1, d0, d1, temb, qp, b, fw1, fb1, fw2, fb2, B16):
    return pl.pallas_call(
        _tc_final_body,
        grid=(NP // BLK,),
        in_specs=[
            pl.BlockSpec((BLK, HID), lambda i: (i, 0)),
            pl.BlockSpec((BLK, HID), lambda i: (i, 0)),
            pl.BlockSpec((BLK, 16), lambda i: (i, 0)),
            pl.BlockSpec((BLK, 16), lambda i: (i, 0)),
            pl.BlockSpec((BLK, HID), lambda i: (i, 0)),
            pl.BlockSpec((BLK, NLABEL), lambda i: (i, 0)),
            pl.BlockSpec((1, HID), lambda i: (0, 0)),
            pl.BlockSpec((HID, 2 * FDIM), lambda i: (0, 0)),
            pl.BlockSpec((NLABEL, 2 * FDIM), lambda i: (0, 0)),
            pl.BlockSpec((2 * FDIM,), lambda i: (0,)),
            pl.BlockSpec((2 * FDIM, NLABEL), lambda i: (0, 0)),
            pl.BlockSpec((NLABEL,), lambda i: (0,)),
            pl.BlockSpec((16, HID), lambda i: (0, 0)),
        ],
        out_specs=pl.BlockSpec((BLK, NLABEL), lambda i: (i, 0)),
        out_shape=jax.ShapeDtypeStruct((NP, NLABEL), jnp.float32),
    )(o0, o1, d0, d1, temb, qp, b.reshape(1, HID), fw1[:HID], fw1[HID:],
      fb1, fw2, fb2, B16)


def kernel(x, q_Y_sample, adj, t, num_steps, W0, asrc0, adst0, b0,
           W1, asrc1, adst1, b1, tw1, tb1, tw2, tb2, fw1, fb1, fw2, fb2):
    f32 = jnp.float32
    xp = jnp.zeros((NP, 128), f32).at[:N].set(x)
    qp = jnp.zeros((NP, NLABEL), f32).at[:N].set(q_Y_sample)
    tsc = jnp.zeros((NP, 1), f32).at[:N, 0].set(t / num_steps * num_steps * 4.0)
    freq = jnp.exp(np.arange(64, dtype=np.float32) * (-(np.log(10000.0) / 63))
                   ).reshape(1, 64).astype(f32)
    hd_of = np.arange(HID) // NHID
    S = np.zeros((HID, 16), np.float32)
    S[np.arange(HID), hd_of] = 1.0
    B16 = jnp.asarray(S.T)
    S = jnp.asarray(S)
    loop = jnp.arange(N, dtype=adj.dtype)
    srcp = jnp.full((EP,), N, jnp.int32).at[:E].set(adj[0]).at[E:ET].set(loop)
    dstp = jnp.full((EP,), N, jnp.int32).at[:E].set(adj[1]).at[E:ET].set(loop)
    srcp = srcp.reshape(NW, NGRP, G, C)
    dstp = dstp.reshape(NW, NGRP, G, C)
    zero = jnp.zeros((NP, PACK), f32)

    temb = _tc_temb(tsc, freq, tw1, tb1, tw2, tb2)
    hal0, ad0 = _tc_pre(xp, qp, W0, asrc0, adst0, S)
    out0, den0 = _make_sc_gat()(hal0, ad0, srcp, dstp, zero)
    hal1, ad1 = _tc_mid(out0[0], out0[1], den0[0], den0[1], temb, qp, b0,
                        W1, asrc1, adst1, S, B16)
    out1, den1 = _make_sc_gat()(hal1, ad1, srcp, dstp, zero)
    out = _tc_final(out1[0], out1[1], den1[0], den1[1], temb, qp, b1,
                    fw1, fb1, fw2, fb2, B16)
    return out[:N]
